# baseline (speedup 1.0000x reference)
import functools

import jax
import jax.numpy as jnp
from jax import lax
from jax.experimental import pallas as pl
from jax.experimental.pallas import tpu as pltpu
from jax.experimental.pallas import tpu_sc as plsc

N = 10000
E = 320000
IN_DIM = 128
HID_DIM = 128
OUT_DIM = 64

NC = 2
NS = 16
NW = NC * NS
CH = 128
NFULL = 80
NPAIR = NFULL // 2
NHALF = NFULL // 2
NPAIR_H = NHALF // 2
E_PAD = NW * NFULL * CH
N_PAD = 10240
RSTRIPE = N_PAD // NS
DEG_MINOR = 4

ROWS_BLK = 400


def _sc_mesh():
    return plsc.VectorSubcoreMesh(core_axis_name="c", subcore_axis_name="s")


def _make_deg_kernel():
    @functools.partial(
        pl.kernel,
        out_type=jax.ShapeDtypeStruct((NC, N_PAD, DEG_MINOR), jnp.float32),
        mesh=_sc_mesh(),
        compiler_params=pltpu.CompilerParams(use_tc_tiling_on_sc=False),
        scratch_types=[
            pltpu.VMEM_SHARED((N_PAD, DEG_MINOR), jnp.float32),
            pltpu.VMEM((CH,), jnp.int32),
            pltpu.VMEM((CH, DEG_MINOR), jnp.float32),
            pltpu.SemaphoreType.DMA,
            pltpu.SemaphoreType.DMA,
        ],
    )
    def deg_kernel(dst1_hbm, z_hbm, ones_hbm, out_hbm, acc, dbuf, ones_v,
                   ssem0, ssem1):
        c = lax.axis_index("c")
        s = lax.axis_index("s")
        wid = c * NS + s
        pltpu.sync_copy(z_hbm.at[pl.ds(s * RSTRIPE, RSTRIPE), :],
                        acc.at[pl.ds(s * RSTRIPE, RSTRIPE), :])
        pltpu.sync_copy(ones_hbm, ones_v)
        plsc.subcore_barrier()
        base = wid * (NFULL * CH)

        def body(j, carry):
            pltpu.sync_copy(dst1_hbm.at[pl.ds(base + j * CH, CH)], dbuf)
            pltpu.async_copy(ones_v, acc.at[dbuf], ssem0,
                             add=True).wait()
            return carry

        lax.fori_loop(0, NFULL, body, 0)
        plsc.subcore_barrier()
        pltpu.sync_copy(acc.at[pl.ds(s * RSTRIPE, RSTRIPE), :],
                        out_hbm.at[c, pl.ds(s * RSTRIPE, RSTRIPE), :])

    return deg_kernel


def _make_scatter_kernel(d):
    @functools.partial(
        pl.kernel,
        out_type=jax.ShapeDtypeStruct((NC, N_PAD, d), jnp.float32),
        mesh=_sc_mesh(),
        compiler_params=pltpu.CompilerParams(use_tc_tiling_on_sc=False),
        scratch_types=[
            pltpu.VMEM_SHARED((N_PAD, d), jnp.float32),
            pltpu.VMEM((CH,), jnp.int32),
            pltpu.VMEM((CH,), jnp.int32),
            pltpu.VMEM((CH, d), jnp.float32),
            pltpu.VMEM((CH, d), jnp.float32),
            pltpu.SemaphoreType.DMA,
            pltpu.SemaphoreType.DMA,
            pltpu.SemaphoreType.DMA,
            pltpu.SemaphoreType.DMA,
        ],
    )
    def scatter_kernel(g_hbm, src1_hbm, dst1_hbm, z_hbm, out_hbm,
                       acc, sbuf, dbuf, rows0, rows1,
                       gsem0, gsem1, ssem0, ssem1):
        c = lax.axis_index("c")
        s = lax.axis_index("s")
        wid = c * NS + s
        pltpu.sync_copy(z_hbm.at[pl.ds(s * RSTRIPE, RSTRIPE), :],
                        acc.at[pl.ds(s * RSTRIPE, RSTRIPE), :])
        plsc.subcore_barrier()

        base = wid * (NFULL * CH)

        def body(j, carry):
            off = base + j * CH
            pltpu.sync_copy(src1_hbm.at[pl.ds(off, CH)], sbuf)
            pltpu.sync_copy(dst1_hbm.at[pl.ds(off, CH)], dbuf)
            pltpu.async_copy(g_hbm.at[sbuf], rows0, gsem0).wait()
            pltpu.async_copy(rows0, acc.at[dbuf], ssem0,
                             add=True).wait()
            return carry

        lax.fori_loop(0, NFULL, body, 0)
        plsc.subcore_barrier()
        pltpu.sync_copy(acc.at[pl.ds(s * RSTRIPE, RSTRIPE), :],
                        out_hbm.at[c, pl.ds(s * RSTRIPE, RSTRIPE), :])

    return scatter_kernel


_deg_call = _make_deg_kernel()
_scatter_hid = _make_scatter_kernel(HID_DIM)
_scatter_out = _make_scatter_kernel(OUT_DIM)


def _mm1_body(x_ref, w_ref, o_ref):
    o_ref[...] = jnp.dot(x_ref[...], w_ref[...], preferred_element_type=jnp.float32)


def _mm1(x, W1):
    grid = N // ROWS_BLK
    return pl.pallas_call(
        _mm1_body,
        grid=(grid,),
        in_specs=[
            pl.BlockSpec((ROWS_BLK, IN_DIM), lambda i: (i, 0)),
            pl.BlockSpec((IN_DIM, HID_DIM), lambda i: (0, 0)),
        ],
        out_specs=pl.BlockSpec((ROWS_BLK, HID_DIM), lambda i: (i, 0)),
        out_shape=jax.ShapeDtypeStruct((N, HID_DIM), jnp.float32),
    )(x, W1)


def _scale_body(deg_ref, m_ref, dinv_ref, g_ref):
    dsum = jnp.sum(deg_ref[...], axis=0)
    deg = dsum[:, 0:1] + 1.0
    dinv = lax.rsqrt(deg)
    dinv_ref[...] = dinv
    g_ref[...] = m_ref[...] * dinv


def _scale(degp, m1):
    grid = N // ROWS_BLK
    return pl.pallas_call(
        _scale_body,
        grid=(grid,),
        in_specs=[
            pl.BlockSpec((NC, ROWS_BLK, DEG_MINOR), lambda i: (0, i, 0)),
            pl.BlockSpec((ROWS_BLK, HID_DIM), lambda i: (i, 0)),
        ],
        out_specs=[
            pl.BlockSpec((ROWS_BLK, 1), lambda i: (i, 0)),
            pl.BlockSpec((ROWS_BLK, HID_DIM), lambda i: (i, 0)),
        ],
        out_shape=[
            jax.ShapeDtypeStruct((N, 1), jnp.float32),
            jax.ShapeDtypeStruct((N_PAD, HID_DIM), jnp.float32),
        ],
    )(degp, m1)


def _combine_mm_body(p_ref, g_ref, dinv_ref, b_ref, w_ref, o_ref):
    dinv = dinv_ref[...]
    s = p_ref[0] + p_ref[1] + g_ref[...]
    h = jnp.maximum(dinv * s + b_ref[...], 0.0)
    m2 = jnp.dot(h, w_ref[...], preferred_element_type=jnp.float32)
    o_ref[...] = m2 * dinv


def _combine_mm(p1, g1, dinv, b1, W2):
    grid = N // ROWS_BLK
    return pl.pallas_call(
        _combine_mm_body,
        grid=(grid,),
        in_specs=[
            pl.BlockSpec((NC, ROWS_BLK, HID_DIM), lambda i: (0, i, 0)),
            pl.BlockSpec((ROWS_BLK, HID_DIM), lambda i: (i, 0)),
            pl.BlockSpec((ROWS_BLK, 1), lambda i: (i, 0)),
            pl.BlockSpec((1, HID_DIM), lambda i: (0, 0)),
            pl.BlockSpec((HID_DIM, OUT_DIM), lambda i: (0, 0)),
        ],
        out_specs=pl.BlockSpec((ROWS_BLK, OUT_DIM), lambda i: (i, 0)),
        out_shape=jax.ShapeDtypeStruct((N_PAD, OUT_DIM), jnp.float32),
    )(p1, g1, dinv, b1, W2)


def _final_body(p_ref, g_ref, dinv_ref, b_ref, o_ref):
    z = dinv_ref[...] * (p_ref[0] + p_ref[1] + g_ref[...]) + b_ref[...]
    zmax = jnp.max(z, axis=1, keepdims=True)
    lse = jnp.log(jnp.sum(jnp.exp(z - zmax), axis=1, keepdims=True))
    o_ref[...] = z - zmax - lse


def _final(p2, g2, dinv, b2):
    grid = N // ROWS_BLK
    return pl.pallas_call(
        _final_body,
        grid=(grid,),
        in_specs=[
            pl.BlockSpec((NC, ROWS_BLK, OUT_DIM), lambda i: (0, i, 0)),
            pl.BlockSpec((ROWS_BLK, OUT_DIM), lambda i: (i, 0)),
            pl.BlockSpec((ROWS_BLK, 1), lambda i: (i, 0)),
            pl.BlockSpec((1, OUT_DIM), lambda i: (0, 0)),
        ],
        out_specs=pl.BlockSpec((ROWS_BLK, OUT_DIM), lambda i: (i, 0)),
        out_shape=jax.ShapeDtypeStruct((N, OUT_DIM), jnp.float32),
    )(p2, g2, dinv, b2)


def kernel(x, edge_index, W1, b1, W2, b2):
    ei = edge_index.astype(jnp.int32)
    pad = jnp.full((2, E_PAD - E), N, jnp.int32)
    ei_p = jnp.concatenate([ei, pad], axis=1)
    src1 = ei_p[0]
    dst1 = ei_p[1]
    z8 = jnp.zeros((N_PAD, DEG_MINOR), jnp.float32)
    z128 = jnp.zeros((N_PAD, HID_DIM), jnp.float32)
    z64 = jnp.zeros((N_PAD, OUT_DIM), jnp.float32)
    ones8 = jnp.ones((CH, DEG_MINOR), jnp.float32)
    b1r = b1.reshape(1, HID_DIM)
    b2r = b2.reshape(1, OUT_DIM)

    degp = _deg_call(dst1, z8, ones8)
    m1 = _mm1(x, W1)
    dinv, g1 = _scale(degp, m1)
    p1 = _scatter_hid(g1, src1, dst1, z128)
    g2 = _combine_mm(p1, g1, dinv, b1r, W2)
    p2 = _scatter_out(g2, src1, dst1, z64)
    return _final(p2, g2, dinv, b2r)


# speedup vs baseline: 1.9579x; 1.9579x over previous
import functools

import jax
import jax.numpy as jnp
from jax import lax
from jax.experimental import pallas as pl
from jax.experimental.pallas import tpu as pltpu
from jax.experimental.pallas import tpu_sc as plsc

N = 10000
E = 320000
IN_DIM = 128
HID_DIM = 128
OUT_DIM = 64

NC = 2
NS = 16
NW = NC * NS
CH = 128
NFULL = 80
NPAIR = NFULL // 2
NHALF = NFULL // 2
NPAIR_H = NHALF // 2
E_PAD = NW * NFULL * CH
N_PAD = 10240
RSTRIPE = N_PAD // NS
DEG_MINOR = 4

ROWS_BLK = 400


def _sc_mesh():
    return plsc.VectorSubcoreMesh(core_axis_name="c", subcore_axis_name="s")


def _make_deg_kernel():
    @functools.partial(
        pl.kernel,
        out_type=jax.ShapeDtypeStruct((NC, N_PAD, DEG_MINOR), jnp.float32),
        mesh=_sc_mesh(),
        compiler_params=pltpu.CompilerParams(use_tc_tiling_on_sc=False),
        scratch_types=[
            pltpu.VMEM_SHARED((N_PAD, DEG_MINOR), jnp.float32),
            pltpu.VMEM((CH,), jnp.int32),
            pltpu.VMEM((CH, DEG_MINOR), jnp.float32),
            pltpu.SemaphoreType.DMA,
            pltpu.SemaphoreType.DMA,
        ],
    )
    def deg_kernel(dst1_hbm, z_hbm, ones_hbm, out_hbm, acc, dbuf, ones_v,
                   ssem0, ssem1):
        c = lax.axis_index("c")
        s = lax.axis_index("s")
        wid = c * NS + s
        pltpu.sync_copy(z_hbm.at[pl.ds(s * RSTRIPE, RSTRIPE), :],
                        acc.at[pl.ds(s * RSTRIPE, RSTRIPE), :])
        pltpu.sync_copy(ones_hbm, ones_v)
        plsc.subcore_barrier()
        base = wid * (NFULL * CH)

        def body(j, carry):
            pltpu.sync_copy(dst1_hbm.at[pl.ds(base + j * CH, CH)], dbuf)
            pltpu.async_copy(ones_v, acc.at[dbuf], ssem0,
                             add=True).wait()
            return carry

        lax.fori_loop(0, NFULL, body, 0)
        plsc.subcore_barrier()
        pltpu.sync_copy(acc.at[pl.ds(s * RSTRIPE, RSTRIPE), :],
                        out_hbm.at[c, pl.ds(s * RSTRIPE, RSTRIPE), :])

    return deg_kernel


def _make_scatter_kernel(d):
    @functools.partial(
        pl.kernel,
        out_type=jax.ShapeDtypeStruct((NC, N_PAD, d), jnp.float32),
        mesh=_sc_mesh(),
        compiler_params=pltpu.CompilerParams(use_tc_tiling_on_sc=False),
        scratch_types=[
            pltpu.VMEM_SHARED((N_PAD, d), jnp.float32),
            pltpu.VMEM((CH,), jnp.int32),
            pltpu.VMEM((CH,), jnp.int32),
            pltpu.VMEM((CH, d), jnp.float32),
            pltpu.VMEM((CH, d), jnp.float32),
            pltpu.SemaphoreType.DMA,
            pltpu.SemaphoreType.DMA,
            pltpu.SemaphoreType.DMA,
            pltpu.SemaphoreType.DMA,
        ],
    )
    def scatter_kernel(g_hbm, src1_hbm, dst1_hbm, z_hbm, out_hbm,
                       acc, sbuf, dbuf, rows0, rows1,
                       gsem0, gsem1, ssem0, ssem1):
        c = lax.axis_index("c")
        s = lax.axis_index("s")
        wid = c * NS + s
        pltpu.sync_copy(z_hbm.at[pl.ds(s * RSTRIPE, RSTRIPE), :],
                        acc.at[pl.ds(s * RSTRIPE, RSTRIPE), :])
        plsc.subcore_barrier()

        base = wid * (NFULL * CH)

        def body(j, carry):
            off = base + j * CH
            pltpu.sync_copy(src1_hbm.at[pl.ds(off, CH)], sbuf)
            pltpu.sync_copy(dst1_hbm.at[pl.ds(off, CH)], dbuf)
            pltpu.async_copy(g_hbm.at[sbuf], rows0, gsem0).wait()
            pltpu.async_copy(rows0, acc.at[dbuf], ssem0,
                             add=True).wait()
            return carry

        lax.fori_loop(0, NFULL, body, 0)
        plsc.subcore_barrier()
        pltpu.sync_copy(acc.at[pl.ds(s * RSTRIPE, RSTRIPE), :],
                        out_hbm.at[c, pl.ds(s * RSTRIPE, RSTRIPE), :])

    return scatter_kernel


_deg_call = _make_deg_kernel()
_scatter_hid = _make_scatter_kernel(HID_DIM)
_scatter_out = _make_scatter_kernel(OUT_DIM)


def _mm1_body(x_ref, w_ref, o_ref):
    o_ref[...] = jnp.dot(x_ref[...], w_ref[...], preferred_element_type=jnp.float32)


def _mm1(x, W1):
    grid = N // ROWS_BLK
    return pl.pallas_call(
        _mm1_body,
        grid=(grid,),
        in_specs=[
            pl.BlockSpec((ROWS_BLK, IN_DIM), lambda i: (i, 0)),
            pl.BlockSpec((IN_DIM, HID_DIM), lambda i: (0, 0)),
        ],
        out_specs=pl.BlockSpec((ROWS_BLK, HID_DIM), lambda i: (i, 0)),
        out_shape=jax.ShapeDtypeStruct((N, HID_DIM), jnp.float32),
    )(x, W1)


def _scale_body(deg_ref, m_ref, dinv_ref, g_ref):
    dsum = jnp.sum(deg_ref[...], axis=0)
    deg = dsum[:, 0:1] + 1.0
    dinv = lax.rsqrt(deg)
    dinv_ref[...] = dinv
    g_ref[...] = m_ref[...] * dinv


def _scale(degp, m1):
    grid = N // ROWS_BLK
    return pl.pallas_call(
        _scale_body,
        grid=(grid,),
        in_specs=[
            pl.BlockSpec((NC, ROWS_BLK, DEG_MINOR), lambda i: (0, i, 0)),
            pl.BlockSpec((ROWS_BLK, HID_DIM), lambda i: (i, 0)),
        ],
        out_specs=[
            pl.BlockSpec((ROWS_BLK, 1), lambda i: (i, 0)),
            pl.BlockSpec((ROWS_BLK, HID_DIM), lambda i: (i, 0)),
        ],
        out_shape=[
            jax.ShapeDtypeStruct((N, 1), jnp.float32),
            jax.ShapeDtypeStruct((N_PAD, HID_DIM), jnp.float32),
        ],
    )(degp, m1)


def _combine_mm_body(p_ref, g_ref, dinv_ref, b_ref, w_ref, o_ref):
    dinv = dinv_ref[...]
    s = p_ref[0] + p_ref[1] + g_ref[...]
    h = jnp.maximum(dinv * s + b_ref[...], 0.0)
    m2 = jnp.dot(h, w_ref[...], preferred_element_type=jnp.float32)
    o_ref[...] = m2 * dinv


def _combine_mm(p1, g1, dinv, b1, W2):
    grid = N // ROWS_BLK
    return pl.pallas_call(
        _combine_mm_body,
        grid=(grid,),
        in_specs=[
            pl.BlockSpec((NC, ROWS_BLK, HID_DIM), lambda i: (0, i, 0)),
            pl.BlockSpec((ROWS_BLK, HID_DIM), lambda i: (i, 0)),
            pl.BlockSpec((ROWS_BLK, 1), lambda i: (i, 0)),
            pl.BlockSpec((1, HID_DIM), lambda i: (0, 0)),
            pl.BlockSpec((HID_DIM, OUT_DIM), lambda i: (0, 0)),
        ],
        out_specs=pl.BlockSpec((ROWS_BLK, OUT_DIM), lambda i: (i, 0)),
        out_shape=jax.ShapeDtypeStruct((N_PAD, OUT_DIM), jnp.float32),
    )(p1, g1, dinv, b1, W2)


def _final_body(p_ref, g_ref, dinv_ref, b_ref, o_ref):
    z = dinv_ref[...] * (p_ref[0] + p_ref[1] + g_ref[...]) + b_ref[...]
    zmax = jnp.max(z, axis=1, keepdims=True)
    lse = jnp.log(jnp.sum(jnp.exp(z - zmax), axis=1, keepdims=True))
    o_ref[...] = z - zmax - lse


def _final(p2, g2, dinv, b2):
    grid = N // ROWS_BLK
    return pl.pallas_call(
        _final_body,
        grid=(grid,),
        in_specs=[
            pl.BlockSpec((NC, ROWS_BLK, OUT_DIM), lambda i: (0, i, 0)),
            pl.BlockSpec((ROWS_BLK, OUT_DIM), lambda i: (i, 0)),
            pl.BlockSpec((ROWS_BLK, 1), lambda i: (i, 0)),
            pl.BlockSpec((1, OUT_DIM), lambda i: (0, 0)),
        ],
        out_specs=pl.BlockSpec((ROWS_BLK, OUT_DIM), lambda i: (i, 0)),
        out_shape=jax.ShapeDtypeStruct((N, OUT_DIM), jnp.float32),
    )(p2, g2, dinv, b2)


def kernel(x, edge_index, W1, b1, W2, b2):
    ei = edge_index.astype(jnp.int32)
    pad_ids = N + jnp.arange(E_PAD - E, dtype=jnp.int32) % (N_PAD - N)
    pad = jnp.stack([pad_ids, pad_ids])
    ei_p = jnp.concatenate([ei, pad], axis=1)
    src1 = ei_p[0]
    dst1 = ei_p[1]
    z8 = jnp.zeros((N_PAD, DEG_MINOR), jnp.float32)
    z128 = jnp.zeros((N_PAD, HID_DIM), jnp.float32)
    z64 = jnp.zeros((N_PAD, OUT_DIM), jnp.float32)
    ones8 = jnp.ones((CH, DEG_MINOR), jnp.float32)
    b1r = b1.reshape(1, HID_DIM)
    b2r = b2.reshape(1, OUT_DIM)

    degp = _deg_call(dst1, z8, ones8)
    m1 = _mm1(x, W1)
    dinv, g1 = _scale(degp, m1)
    p1 = _scatter_hid(g1, src1, dst1, z128)
    g2 = _combine_mm(p1, g1, dinv, b1r, W2)
    p2 = _scatter_out(g2, src1, dst1, z64)
    return _final(p2, g2, dinv, b2r)


# speedup vs baseline: 2.7258x; 1.3922x over previous
import functools

import jax
import jax.numpy as jnp
from jax import lax
from jax.experimental import pallas as pl
from jax.experimental.pallas import tpu as pltpu
from jax.experimental.pallas import tpu_sc as plsc

N = 10000
E = 320000
IN_DIM = 128
HID_DIM = 128
OUT_DIM = 64

NC = 2
NS = 16
NW = NC * NS
CH = 128
NFULL = 80
NPAIR = NFULL // 2
NHALF = NFULL // 2
NPAIR_H = NHALF // 2
E_PAD = NW * NFULL * CH
N_PAD = 10240
RSTRIPE = N_PAD // NS
DEG_MINOR = 4

ROWS_BLK = 400


def _sc_mesh():
    return plsc.VectorSubcoreMesh(core_axis_name="c", subcore_axis_name="s")


def _make_deg_kernel():
    @functools.partial(
        pl.kernel,
        out_type=jax.ShapeDtypeStruct((NC, N_PAD, DEG_MINOR), jnp.float32),
        mesh=_sc_mesh(),
        compiler_params=pltpu.CompilerParams(use_tc_tiling_on_sc=False),
        scratch_types=[
            pltpu.VMEM_SHARED((N_PAD, DEG_MINOR), jnp.float32),
            pltpu.VMEM((CH,), jnp.int32),
            pltpu.VMEM((CH, DEG_MINOR), jnp.float32),
            pltpu.SemaphoreType.DMA,
            pltpu.SemaphoreType.DMA,
        ],
    )
    def deg_kernel(dst1_hbm, z_hbm, ones_hbm, out_hbm, acc, dbuf, ones_v,
                   ssem0, ssem1):
        c = lax.axis_index("c")
        s = lax.axis_index("s")
        wid = c * NS + s
        pltpu.sync_copy(z_hbm.at[pl.ds(s * RSTRIPE, RSTRIPE), :],
                        acc.at[pl.ds(s * RSTRIPE, RSTRIPE), :])
        pltpu.sync_copy(ones_hbm, ones_v)
        plsc.subcore_barrier()
        base = wid * (NFULL * CH)

        def body(j, carry):
            pltpu.sync_copy(dst1_hbm.at[pl.ds(base + j * CH, CH)], dbuf)
            pltpu.async_copy(ones_v, acc.at[dbuf], ssem0,
                             add=True).wait()
            return carry

        lax.fori_loop(0, NFULL, body, 0)
        plsc.subcore_barrier()
        pltpu.sync_copy(acc.at[pl.ds(s * RSTRIPE, RSTRIPE), :],
                        out_hbm.at[c, pl.ds(s * RSTRIPE, RSTRIPE), :])

    return deg_kernel


def _make_scatter_kernel(d, K):
    @functools.partial(
        pl.kernel,
        out_type=jax.ShapeDtypeStruct((NC, N_PAD, d), jnp.float32),
        mesh=_sc_mesh(),
        compiler_params=pltpu.CompilerParams(use_tc_tiling_on_sc=False),
        scratch_types=(
            [pltpu.VMEM_SHARED((N_PAD, d), jnp.float32)]
            + [pltpu.VMEM((CH,), jnp.int32)] * (2 * K)
            + [pltpu.VMEM((CH, d), jnp.float32)] * K
            + [pltpu.SemaphoreType.DMA] * 3
        ),
    )
    def scatter_kernel(g_hbm, src1_hbm, dst1_hbm, z_hbm, out_hbm, *rest):
        acc = rest[0]
        sbufs = rest[1:1 + K]
        dbufs = rest[1 + K:1 + 2 * K]
        rows = rest[1 + 2 * K:1 + 3 * K]
        isem, gsem, ssem = rest[1 + 3 * K:]
        c = lax.axis_index("c")
        s = lax.axis_index("s")
        wid = c * NS + s
        pltpu.sync_copy(z_hbm.at[pl.ds(s * RSTRIPE, RSTRIPE), :],
                        acc.at[pl.ds(s * RSTRIPE, RSTRIPE), :])
        plsc.subcore_barrier()

        base = wid * (NFULL * CH)

        def body(gi, carry):
            a = gi * K
            loads = []
            for k in range(K):
                off = base + (a + k) * CH
                loads.append(pltpu.async_copy(
                    src1_hbm.at[pl.ds(off, CH)], sbufs[k], isem))
                loads.append(pltpu.async_copy(
                    dst1_hbm.at[pl.ds(off, CH)], dbufs[k], isem))
            for ld in loads:
                ld.wait()
            gats = [pltpu.async_copy(g_hbm.at[sbufs[k]], rows[k], gsem)
                    for k in range(K)]
            for g in gats:
                g.wait()
            for k in range(K):
                pltpu.async_copy(rows[k], acc.at[dbufs[k]], ssem,
                                 add=True).wait()
            return carry

        lax.fori_loop(0, NFULL // K, body, 0)
        plsc.subcore_barrier()
        pltpu.sync_copy(acc.at[pl.ds(s * RSTRIPE, RSTRIPE), :],
                        out_hbm.at[c, pl.ds(s * RSTRIPE, RSTRIPE), :])

    return scatter_kernel


_deg_call = _make_deg_kernel()
_scatter_hid = _make_scatter_kernel(HID_DIM, 2)
_scatter_out = _make_scatter_kernel(OUT_DIM, 4)


def _mm1_body(x_ref, w_ref, o_ref):
    o_ref[...] = jnp.dot(x_ref[...], w_ref[...], preferred_element_type=jnp.float32)


def _mm1(x, W1):
    grid = N // ROWS_BLK
    return pl.pallas_call(
        _mm1_body,
        grid=(grid,),
        in_specs=[
            pl.BlockSpec((ROWS_BLK, IN_DIM), lambda i: (i, 0)),
            pl.BlockSpec((IN_DIM, HID_DIM), lambda i: (0, 0)),
        ],
        out_specs=pl.BlockSpec((ROWS_BLK, HID_DIM), lambda i: (i, 0)),
        out_shape=jax.ShapeDtypeStruct((N, HID_DIM), jnp.float32),
    )(x, W1)


def _scale_body(deg_ref, m_ref, dinv_ref, g_ref):
    dsum = jnp.sum(deg_ref[...], axis=0)
    deg = dsum[:, 0:1] + 1.0
    dinv = lax.rsqrt(deg)
    dinv_ref[...] = dinv
    g_ref[...] = m_ref[...] * dinv


def _scale(degp, m1):
    grid = N // ROWS_BLK
    return pl.pallas_call(
        _scale_body,
        grid=(grid,),
        in_specs=[
            pl.BlockSpec((NC, ROWS_BLK, DEG_MINOR), lambda i: (0, i, 0)),
            pl.BlockSpec((ROWS_BLK, HID_DIM), lambda i: (i, 0)),
        ],
        out_specs=[
            pl.BlockSpec((ROWS_BLK, 1), lambda i: (i, 0)),
            pl.BlockSpec((ROWS_BLK, HID_DIM), lambda i: (i, 0)),
        ],
        out_shape=[
            jax.ShapeDtypeStruct((N, 1), jnp.float32),
            jax.ShapeDtypeStruct((N_PAD, HID_DIM), jnp.float32),
        ],
    )(degp, m1)


def _combine_mm_body(p_ref, g_ref, dinv_ref, b_ref, w_ref, o_ref):
    dinv = dinv_ref[...]
    s = p_ref[0] + p_ref[1] + g_ref[...]
    h = jnp.maximum(dinv * s + b_ref[...], 0.0)
    m2 = jnp.dot(h, w_ref[...], preferred_element_type=jnp.float32)
    o_ref[...] = m2 * dinv


def _combine_mm(p1, g1, dinv, b1, W2):
    grid = N // ROWS_BLK
    return pl.pallas_call(
        _combine_mm_body,
        grid=(grid,),
        in_specs=[
            pl.BlockSpec((NC, ROWS_BLK, HID_DIM), lambda i: (0, i, 0)),
            pl.BlockSpec((ROWS_BLK, HID_DIM), lambda i: (i, 0)),
            pl.BlockSpec((ROWS_BLK, 1), lambda i: (i, 0)),
            pl.BlockSpec((1, HID_DIM), lambda i: (0, 0)),
            pl.BlockSpec((HID_DIM, OUT_DIM), lambda i: (0, 0)),
        ],
        out_specs=pl.BlockSpec((ROWS_BLK, OUT_DIM), lambda i: (i, 0)),
        out_shape=jax.ShapeDtypeStruct((N_PAD, OUT_DIM), jnp.float32),
    )(p1, g1, dinv, b1, W2)


def _final_body(p_ref, g_ref, dinv_ref, b_ref, o_ref):
    z = dinv_ref[...] * (p_ref[0] + p_ref[1] + g_ref[...]) + b_ref[...]
    zmax = jnp.max(z, axis=1, keepdims=True)
    lse = jnp.log(jnp.sum(jnp.exp(z - zmax), axis=1, keepdims=True))
    o_ref[...] = z - zmax - lse


def _final(p2, g2, dinv, b2):
    grid = N // ROWS_BLK
    return pl.pallas_call(
        _final_body,
        grid=(grid,),
        in_specs=[
            pl.BlockSpec((NC, ROWS_BLK, OUT_DIM), lambda i: (0, i, 0)),
            pl.BlockSpec((ROWS_BLK, OUT_DIM), lambda i: (i, 0)),
            pl.BlockSpec((ROWS_BLK, 1), lambda i: (i, 0)),
            pl.BlockSpec((1, OUT_DIM), lambda i: (0, 0)),
        ],
        out_specs=pl.BlockSpec((ROWS_BLK, OUT_DIM), lambda i: (i, 0)),
        out_shape=jax.ShapeDtypeStruct((N, OUT_DIM), jnp.float32),
    )(p2, g2, dinv, b2)


def kernel(x, edge_index, W1, b1, W2, b2):
    ei = edge_index.astype(jnp.int32)
    pad_ids = N + jnp.arange(E_PAD - E, dtype=jnp.int32) % (N_PAD - N)
    pad = jnp.stack([pad_ids, pad_ids])
    ei_p = jnp.concatenate([ei, pad], axis=1)
    src1 = ei_p[0]
    dst1 = ei_p[1]
    z8 = jnp.zeros((N_PAD, DEG_MINOR), jnp.float32)
    z128 = jnp.zeros((N_PAD, HID_DIM), jnp.float32)
    z64 = jnp.zeros((N_PAD, OUT_DIM), jnp.float32)
    ones8 = jnp.ones((CH, DEG_MINOR), jnp.float32)
    b1r = b1.reshape(1, HID_DIM)
    b2r = b2.reshape(1, OUT_DIM)

    degp = _deg_call(dst1, z8, ones8)
    m1 = _mm1(x, W1)
    dinv, g1 = _scale(degp, m1)
    p1 = _scatter_hid(g1, src1, dst1, z128)
    g2 = _combine_mm(p1, g1, dinv, b1r, W2)
    p2 = _scatter_out(g2, src1, dst1, z64)
    return _final(p2, g2, dinv, b2r)
